# baseline (speedup 1.0000x reference)
import functools

import jax
import jax.numpy as jnp
from jax.experimental import pallas as pl
from jax.experimental.pallas import tpu as pltpu


def _rvq_kernel(x_ref, cb_ref, out_ref, loss_ref,
                res_ref, qacc_ref, hi_ref, mid_ref, lo_ref, *, K, TILE):
    q = pl.program_id(0)
    i = pl.program_id(1)

    @pl.when(jnp.logical_and(q == 0, i == 0))
    def _():
        loss_ref[:, :] = jnp.zeros((1, 1), jnp.float32)

    @pl.when(q == 0)
    def _():
        res_ref[i] = x_ref[0]
        qacc_ref[i] = jnp.zeros_like(qacc_ref[i])

    cb = cb_ref[0]

    @pl.when(i == 0)
    def _():
        hi = cb.astype(jnp.bfloat16)
        rem = cb - hi.astype(jnp.float32)
        mid = rem.astype(jnp.bfloat16)
        lo = (rem - mid.astype(jnp.float32)).astype(jnp.bfloat16)
        hi_ref[:, :] = hi
        mid_ref[:, :] = mid
        lo_ref[:, :] = lo

    r = res_ref[i]
    cn = jnp.sum(cb * cb, axis=1, keepdims=True)
    mm = jax.lax.dot_general(
        cb, r, (((1,), (0,)), ((), ())),
        preferred_element_type=jnp.float32)
    s = cn - 2.0 * mm
    iota = jax.lax.broadcasted_iota(jnp.int32, (K, TILE), 0)
    idx = jnp.argmin(s, axis=0).reshape(1, TILE)
    onehot = (iota == idx).astype(jnp.bfloat16)
    qhi = jax.lax.dot_general(
        hi_ref[:, :], onehot, (((0,), (0,)), ((), ())),
        preferred_element_type=jnp.float32)
    qmid = jax.lax.dot_general(
        mid_ref[:, :], onehot, (((0,), (0,)), ((), ())),
        preferred_element_type=jnp.float32)
    qlo = jax.lax.dot_general(
        lo_ref[:, :], onehot, (((0,), (0,)), ((), ())),
        preferred_element_type=jnp.float32)
    quant = (qhi + qmid) + qlo
    rnew = r - quant
    res_ref[i] = rnew
    qacc_ref[i] = qacc_ref[i] + quant
    loss_ref[:, :] += jnp.sum(rnew * rnew).reshape(1, 1)
    out_ref[0] = qacc_ref[i]


def kernel(x, codebooks):
    B, D, T = x.shape
    Q, K, _ = codebooks.shape
    TILE = 256 if T % 256 == 0 else T
    tiles_per_b = T // TILE
    num_tiles = B * tiles_per_b
    grid = (Q, num_tiles)

    quant, loss = pl.pallas_call(
        functools.partial(_rvq_kernel, K=K, TILE=TILE),
        grid=grid,
        in_specs=[
            pl.BlockSpec((1, D, TILE), lambda q, i: (i // tiles_per_b, 0, i % tiles_per_b)),
            pl.BlockSpec((1, K, D), lambda q, i: (q, 0, 0)),
        ],
        out_specs=[
            pl.BlockSpec((1, D, TILE), lambda q, i: (i // tiles_per_b, 0, i % tiles_per_b)),
            pl.BlockSpec((1, 1), lambda q, i: (0, 0)),
        ],
        out_shape=[
            jax.ShapeDtypeStruct((B, D, T), jnp.float32),
            jax.ShapeDtypeStruct((1, 1), jnp.float32),
        ],
        scratch_shapes=[
            pltpu.VMEM((num_tiles, D, TILE), jnp.float32),
            pltpu.VMEM((num_tiles, D, TILE), jnp.float32),
            pltpu.VMEM((K, D), jnp.bfloat16),
            pltpu.VMEM((K, D), jnp.bfloat16),
            pltpu.VMEM((K, D), jnp.bfloat16),
        ],
    )(x, codebooks)
    return quant, (loss[0, 0] / (B * T * D)).astype(jnp.float32)


# speedup vs baseline: 2.5966x; 2.5966x over previous
import functools

import jax
import jax.numpy as jnp
from jax.experimental import pallas as pl
from jax.experimental.pallas import tpu as pltpu


def _rvq_kernel(x_ref, cb_ref, out_ref, loss_ref,
                res_ref, split_ref, *, K, TILE):
    q = pl.program_id(0)
    i = pl.program_id(1)

    @pl.when(jnp.logical_and(q == 0, i == 0))
    def _():
        loss_ref[:, :] = jnp.zeros((1, 1), jnp.float32)

    @pl.when(q == 0)
    def _():
        res_ref[i] = x_ref[0]

    cb = cb_ref[0]

    @pl.when(i == 0)
    def _():
        hi = cb.astype(jnp.bfloat16)
        rem = cb - hi.astype(jnp.float32)
        mid = rem.astype(jnp.bfloat16)
        lo = (rem - mid.astype(jnp.float32)).astype(jnp.bfloat16)
        split_ref[:, :] = jnp.concatenate([hi, mid, lo], axis=1)

    r = res_ref[i]
    cn = jnp.sum(cb * cb, axis=1, keepdims=True)
    mm = jax.lax.dot_general(
        cb, r, (((1,), (0,)), ((), ())),
        preferred_element_type=jnp.float32)
    s = cn - 2.0 * mm
    iota = jax.lax.broadcasted_iota(jnp.int32, (K, TILE), 0)
    idx = jnp.argmin(s, axis=0).reshape(1, TILE)
    onehot = (iota == idx).astype(jnp.bfloat16)
    q3 = jax.lax.dot_general(
        split_ref[:, :], onehot, (((0,), (0,)), ((), ())),
        preferred_element_type=jnp.float32)
    D = cb.shape[1]
    quant = (q3[0:D] + q3[D:2 * D]) + q3[2 * D:3 * D]
    rnew = r - quant
    res_ref[i] = rnew
    loss_ref[:, :] += jnp.sum(rnew * rnew).reshape(1, 1)
    out_ref[0] = x_ref[0] - rnew


def kernel(x, codebooks):
    B, D, T = x.shape
    Q, K, _ = codebooks.shape
    TILE = 2048 if T % 2048 == 0 else T
    tiles_per_b = T // TILE
    num_tiles = B * tiles_per_b
    grid = (Q, num_tiles)

    quant, loss = pl.pallas_call(
        functools.partial(_rvq_kernel, K=K, TILE=TILE),
        grid=grid,
        in_specs=[
            pl.BlockSpec((1, D, TILE), lambda q, i: (i // tiles_per_b, 0, i % tiles_per_b)),
            pl.BlockSpec((1, K, D), lambda q, i: (q, 0, 0)),
        ],
        out_specs=[
            pl.BlockSpec((1, D, TILE), lambda q, i: (i // tiles_per_b, 0, i % tiles_per_b)),
            pl.BlockSpec((1, 1), lambda q, i: (0, 0)),
        ],
        out_shape=[
            jax.ShapeDtypeStruct((B, D, T), jnp.float32),
            jax.ShapeDtypeStruct((1, 1), jnp.float32),
        ],
        scratch_shapes=[
            pltpu.VMEM((num_tiles, D, TILE), jnp.float32),
            pltpu.VMEM((K, 3 * D), jnp.bfloat16),
        ],
    )(x, codebooks)
    return quant, (loss[0, 0] / (B * T * D)).astype(jnp.float32)
